# baseline (speedup 1.0000x reference)
import functools

import jax
import jax.numpy as jnp
from jax import lax
from jax.experimental import pallas as pl
from jax.experimental.pallas import tpu as pltpu
from jax.experimental.pallas import tpu_sc as plsc

_N = 50000
_NW = 32
_WPT = 1568
_LEVELS = (
    (56, 128, 0.25, 64),
    (28, 128, 0.125, 32),
    (14, 256, 0.0625, 16),
    (7, 512, 1.0 / 32.0, 16),
)


def _scratch_types():
    t = [
        pltpu.VMEM((_WPT,), jnp.float32),
        pltpu.VMEM((_WPT,), jnp.float32),
        pltpu.VMEM((_WPT,), jnp.float32),
        pltpu.VMEM((_WPT,), jnp.float32),
        pltpu.VMEM((_WPT,), jnp.float32),
    ]
    for (_, dim, _, blk) in _LEVELS:
        t.extend([pltpu.VMEM((blk, dim), jnp.float32)] * 4)
        t.extend([pltpu.VMEM((blk,), jnp.int32)] * 4)
        t.extend([pltpu.VMEM((blk,), jnp.float32)] * 4)
    t.append(pltpu.SemaphoreType.DMA)
    return t


def _worker_id():
    return lax.axis_index("s") * 2 + lax.axis_index("c")


def _sc_body(x_hbm, y_hbm, z_hbm, t0, t1, t2, t3, o0, o1, o2, o3, *scr):
    xv, yv, zv, hv, wv = scr[:5]
    per_level = []
    k = 5
    for _ in _LEVELS:
        per_level.append(scr[k:k + 12])
        k += 12
    sem = scr[k]

    wid = _worker_id()
    base = jnp.minimum(wid * _WPT, _N - _WPT)

    pltpu.sync_copy(x_hbm.at[pl.ds(base, _WPT)], xv)
    pltpu.sync_copy(y_hbm.at[pl.ds(base, _WPT)], yv)
    pltpu.sync_copy(z_hbm.at[pl.ds(base, _WPT)], zv)

    def hw_body(c, carry):
        s = c * 16
        xx = xv[pl.ds(s, 16)]
        yy = yv[pl.ds(s, 16)]
        zz = zv[pl.ds(s, 16)]
        nz = -zz
        hh = 250.0 * (-yy) / nz + 112.0
        ww = 250.0 * xx / nz + 112.0
        hv[pl.ds(s, 16)] = jnp.minimum(jnp.maximum(hh, 0.0), 223.0)
        wv[pl.ds(s, 16)] = jnp.minimum(jnp.maximum(ww, 0.0), 223.0)
        return carry

    lax.fori_loop(0, _WPT // 16, hw_body, 0)

    tabs = (t0, t1, t2, t3)
    outs = (o0, o1, o2, o3)
    for lvl, (H, dim, scale, blk) in enumerate(_LEVELS):
        tab, out = tabs[lvl], outs[lvl]
        q11, q21, q12, q22, i11, i21, i12, i22, w11, w21, w12, w22 = \
            per_level[lvl]
        nb = -(-_WPT // blk)

        def blk_body(b, carry, tab=tab, out=out, H=H, dim=dim, scale=scale,
                     blk=blk, q11=q11, q21=q21, q12=q12, q22=q22,
                     i11=i11, i21=i21, i12=i12, i22=i22,
                     w11=w11, w21=w21, w12=w12, w22=w22):
            pb = jnp.minimum(b * blk, _WPT - blk)

            def iw_body(c, carry2):
                s = pb + c * 16
                t = c * 16
                hx = hv[pl.ds(s, 16)] * scale
                wy = wv[pl.ds(s, 16)] * scale
                x1i = hx.astype(jnp.int32)
                x1f = x1i.astype(jnp.float32)
                x2f = jnp.where(x1f == hx, x1f, x1f + 1.0)
                x2i = jnp.minimum(x2f.astype(jnp.int32), H - 1)
                y1i = wy.astype(jnp.int32)
                y1f = y1i.astype(jnp.float32)
                y2f = jnp.where(y1f == wy, y1f, y1f + 1.0)
                y2i = jnp.minimum(y2f.astype(jnp.int32), H - 1)
                dx2 = x2f - hx
                dx1 = hx - x1f
                dy2 = y2f - wy
                dy1 = wy - y1f
                i11[pl.ds(t, 16)] = x1i * H + y1i
                i21[pl.ds(t, 16)] = x2i * H + y1i
                i12[pl.ds(t, 16)] = x1i * H + y2i
                i22[pl.ds(t, 16)] = x2i * H + y2i
                w11[pl.ds(t, 16)] = dx2 * dy2
                w21[pl.ds(t, 16)] = dx1 * dy2
                w12[pl.ds(t, 16)] = dx2 * dy1
                w22[pl.ds(t, 16)] = dx1 * dy1
                return carry2

            lax.fori_loop(0, blk // 16, iw_body, 0)

            c1 = pltpu.async_copy(tab.at[i11], q11, sem)
            c2 = pltpu.async_copy(tab.at[i21], q21, sem)
            c3 = pltpu.async_copy(tab.at[i12], q12, sem)
            c4 = pltpu.async_copy(tab.at[i22], q22, sem)
            c1.wait()
            c2.wait()
            c3.wait()
            c4.wait()

            def fma_body(g, carry2):
                p0 = g * 16
                a16 = w11[pl.ds(p0, 16)]
                b16 = w21[pl.ds(p0, 16)]
                c16 = w12[pl.ds(p0, 16)]
                d16 = w22[pl.ds(p0, 16)]
                for j in range(16):
                    p = p0 + j
                    a, bw, cw, dw = a16[j], b16[j], c16[j], d16[j]

                    def ch_body(kk, carry3, p=p, a=a, bw=bw, cw=cw, dw=dw):
                        d = pl.ds(kk * 16, 16)
                        q11[p, d] = (a * q11[p, d] + bw * q21[p, d]
                                     + cw * q12[p, d] + dw * q22[p, d])
                        return carry3

                    lax.fori_loop(0, dim // 16, ch_body, 0, unroll=4)
                return carry2

            lax.fori_loop(0, blk // 16, fma_body, 0)

            pltpu.sync_copy(q11, out.at[pl.ds(base + pb, blk)])
            return carry

        lax.fori_loop(0, nb, blk_body, 0)


@functools.cache
def _build_sc_kernel():
    mesh = plsc.VectorSubcoreMesh(
        core_axis_name="c", subcore_axis_name="s", num_cores=2, num_subcores=16
    )
    return functools.partial(
        pl.kernel,
        out_type=(
            jax.ShapeDtypeStruct((_N, 128), jnp.float32),
            jax.ShapeDtypeStruct((_N, 128), jnp.float32),
            jax.ShapeDtypeStruct((_N, 256), jnp.float32),
            jax.ShapeDtypeStruct((_N, 512), jnp.float32),
        ),
        mesh=mesh,
        scratch_types=_scratch_types(),
    )(_sc_body)


def kernel(inputs, img_feat0, img_feat1, img_feat2, img_feat3):
    x = inputs[:, 0]
    y = inputs[:, 1]
    z = inputs[:, 2]
    t0 = jnp.pad(img_feat0.reshape(56 * 56, 64), ((0, 0), (0, 64)))
    t1 = img_feat1.reshape(28 * 28, 128)
    t2 = img_feat2.reshape(14 * 14, 256)
    t3 = img_feat3.reshape(7 * 7, 512)
    o0, o1, o2, o3 = _build_sc_kernel()(x, y, z, t0, t1, t2, t3)
    return jnp.concatenate([inputs, o0[:, :64], o1, o2, o3], axis=1)


# speedup vs baseline: 1.0330x; 1.0330x over previous
import functools

import jax
import jax.numpy as jnp
from jax import lax
from jax.experimental import pallas as pl
from jax.experimental.pallas import tpu as pltpu
from jax.experimental.pallas import tpu_sc as plsc

_N = 50000
_NW = 32
_WPT = 1568
_LEVELS = (
    (56, 64, 0.25, 64),
    (28, 128, 0.125, 32),
    (14, 256, 0.0625, 16),
    (7, 512, 1.0 / 32.0, 16),
)


def _scratch_types():
    t = [
        pltpu.VMEM((_WPT,), jnp.float32),
        pltpu.VMEM((_WPT,), jnp.float32),
        pltpu.VMEM((_WPT,), jnp.float32),
        pltpu.VMEM((_WPT,), jnp.float32),
        pltpu.VMEM((_WPT,), jnp.float32),
    ]
    for (_, dim, _, blk) in _LEVELS:
        t.extend([pltpu.VMEM((blk, dim), jnp.float32)] * 4)
        t.extend([pltpu.VMEM((blk,), jnp.int32)] * 4)
        t.extend([pltpu.VMEM((blk,), jnp.float32)] * 4)
    t.append(pltpu.SemaphoreType.DMA)
    return t


def _worker_id():
    return lax.axis_index("s") * 2 + lax.axis_index("c")


def _sc_body(x_hbm, y_hbm, z_hbm, t0, t1, t2, t3, o0, o1, o2, o3, *scr):
    xv, yv, zv, hv, wv = scr[:5]
    per_level = []
    k = 5
    for _ in _LEVELS:
        per_level.append(scr[k:k + 12])
        k += 12
    sem = scr[k]

    wid = _worker_id()
    base = jnp.minimum(wid * _WPT, _N - _WPT)

    pltpu.sync_copy(x_hbm.at[pl.ds(base, _WPT)], xv)
    pltpu.sync_copy(y_hbm.at[pl.ds(base, _WPT)], yv)
    pltpu.sync_copy(z_hbm.at[pl.ds(base, _WPT)], zv)

    def hw_body(c, carry):
        s = c * 16
        xx = xv[pl.ds(s, 16)]
        yy = yv[pl.ds(s, 16)]
        zz = zv[pl.ds(s, 16)]
        nz = -zz
        hh = 250.0 * (-yy) / nz + 112.0
        ww = 250.0 * xx / nz + 112.0
        hv[pl.ds(s, 16)] = jnp.minimum(jnp.maximum(hh, 0.0), 223.0)
        wv[pl.ds(s, 16)] = jnp.minimum(jnp.maximum(ww, 0.0), 223.0)
        return carry

    lax.fori_loop(0, _WPT // 16, hw_body, 0)

    tabs = (t0, t1, t2, t3)
    outs = (o0, o1, o2, o3)
    for lvl, (H, dim, scale, blk) in enumerate(_LEVELS):
        tab, out = tabs[lvl], outs[lvl]
        q11, q21, q12, q22, i11, i21, i12, i22, w11, w21, w12, w22 = \
            per_level[lvl]
        nb = -(-_WPT // blk)

        def blk_body(b, carry, tab=tab, out=out, H=H, dim=dim, scale=scale,
                     blk=blk, q11=q11, q21=q21, q12=q12, q22=q22,
                     i11=i11, i21=i21, i12=i12, i22=i22,
                     w11=w11, w21=w21, w12=w12, w22=w22):
            pb = jnp.minimum(b * blk, _WPT - blk)

            def iw_body(c, carry2):
                s = pb + c * 16
                t = c * 16
                hx = hv[pl.ds(s, 16)] * scale
                wy = wv[pl.ds(s, 16)] * scale
                x1i = hx.astype(jnp.int32)
                x1f = x1i.astype(jnp.float32)
                x2f = jnp.where(x1f == hx, x1f, x1f + 1.0)
                x2i = jnp.minimum(x2f.astype(jnp.int32), H - 1)
                y1i = wy.astype(jnp.int32)
                y1f = y1i.astype(jnp.float32)
                y2f = jnp.where(y1f == wy, y1f, y1f + 1.0)
                y2i = jnp.minimum(y2f.astype(jnp.int32), H - 1)
                dx2 = x2f - hx
                dx1 = hx - x1f
                dy2 = y2f - wy
                dy1 = wy - y1f
                i11[pl.ds(t, 16)] = x1i * H + y1i
                i21[pl.ds(t, 16)] = x2i * H + y1i
                i12[pl.ds(t, 16)] = x1i * H + y2i
                i22[pl.ds(t, 16)] = x2i * H + y2i
                w11[pl.ds(t, 16)] = dx2 * dy2
                w21[pl.ds(t, 16)] = dx1 * dy2
                w12[pl.ds(t, 16)] = dx2 * dy1
                w22[pl.ds(t, 16)] = dx1 * dy1
                return carry2

            lax.fori_loop(0, blk // 16, iw_body, 0)

            c1 = pltpu.async_copy(tab.at[i11], q11, sem)
            c2 = pltpu.async_copy(tab.at[i21], q21, sem)
            c3 = pltpu.async_copy(tab.at[i12], q12, sem)
            c4 = pltpu.async_copy(tab.at[i22], q22, sem)
            c1.wait()
            c2.wait()
            c3.wait()
            c4.wait()

            def fma_body(g, carry2):
                p0 = g * 16
                a16 = w11[pl.ds(p0, 16)]
                b16 = w21[pl.ds(p0, 16)]
                c16 = w12[pl.ds(p0, 16)]
                d16 = w22[pl.ds(p0, 16)]
                for j in range(16):
                    p = p0 + j
                    a, bw, cw, dw = a16[j], b16[j], c16[j], d16[j]

                    def ch_body(kk, carry3, p=p, a=a, bw=bw, cw=cw, dw=dw):
                        d = pl.ds(kk * 16, 16)
                        q11[p, d] = (a * q11[p, d] + bw * q21[p, d]
                                     + cw * q12[p, d] + dw * q22[p, d])
                        return carry3

                    lax.fori_loop(0, dim // 16, ch_body, 0, unroll=4)
                return carry2

            lax.fori_loop(0, blk // 16, fma_body, 0)

            pltpu.sync_copy(q11, out.at[pl.ds(base + pb, blk)])
            return carry

        lax.fori_loop(0, nb, blk_body, 0)


@functools.cache
def _build_sc_kernel():
    mesh = plsc.VectorSubcoreMesh(
        core_axis_name="c", subcore_axis_name="s", num_cores=2, num_subcores=16
    )
    return functools.partial(
        pl.kernel,
        out_type=(
            jax.ShapeDtypeStruct((_N, 64), jnp.float32),
            jax.ShapeDtypeStruct((_N, 128), jnp.float32),
            jax.ShapeDtypeStruct((_N, 256), jnp.float32),
            jax.ShapeDtypeStruct((_N, 512), jnp.float32),
        ),
        mesh=mesh,
        scratch_types=_scratch_types(),
        compiler_params=pltpu.CompilerParams(use_tc_tiling_on_sc=False),
    )(_sc_body)


def kernel(inputs, img_feat0, img_feat1, img_feat2, img_feat3):
    x = inputs[:, 0]
    y = inputs[:, 1]
    z = inputs[:, 2]
    t0 = img_feat0.reshape(56 * 56, 64)
    t1 = img_feat1.reshape(28 * 28, 128)
    t2 = img_feat2.reshape(14 * 14, 256)
    t3 = img_feat3.reshape(7 * 7, 512)
    o0, o1, o2, o3 = _build_sc_kernel()(x, y, z, t0, t1, t2, t3)
    return jnp.concatenate([inputs, o0, o1, o2, o3], axis=1)


# speedup vs baseline: 1.7791x; 1.7223x over previous
import functools

import jax
import jax.numpy as jnp
from jax import lax
from jax.experimental import pallas as pl
from jax.experimental.pallas import tpu as pltpu
from jax.experimental.pallas import tpu_sc as plsc

_N = 50000
_NW = 32
_WPT = 1568
_BLK = 16
_OUTD = 963
_LEVELS = (
    (56, 64, 0.25, 3),
    (28, 128, 0.125, 67),
    (14, 256, 0.0625, 195),
    (7, 512, 1.0 / 32.0, 451),
)


def _scratch_types():
    return [
        pltpu.VMEM((_WPT,), jnp.float32),
        pltpu.VMEM((_WPT,), jnp.float32),
        pltpu.VMEM((_WPT,), jnp.float32),
        pltpu.VMEM((_WPT,), jnp.float32),
        pltpu.VMEM((_WPT,), jnp.float32),
    ] + [
        pltpu.VMEM((_BLK, dim), jnp.float32)
        for (_, dim, _, _) in _LEVELS for _c in range(4)
    ] + [
        pltpu.VMEM((_BLK,), jnp.int32),
        pltpu.VMEM((_BLK,), jnp.int32),
        pltpu.VMEM((_BLK,), jnp.int32),
        pltpu.VMEM((_BLK,), jnp.int32),
        pltpu.VMEM((_BLK * _OUTD,), jnp.float32),
        pltpu.SemaphoreType.DMA,
    ]


def _worker_id():
    return lax.axis_index("s") * 2 + lax.axis_index("c")


def _sc_body(x_hbm, y_hbm, z_hbm, t0, t1, t2, t3, out, *scr):
    xv, yv, zv, hv, wv = scr[:5]
    qbufs = [scr[5 + 4 * l:5 + 4 * l + 4] for l in range(4)]
    i11, i21, i12, i22, ob, sem = scr[21:]

    wid = _worker_id()
    base = jnp.minimum(wid * _WPT, _N - _WPT)

    pltpu.sync_copy(x_hbm.at[pl.ds(base, _WPT)], xv)
    pltpu.sync_copy(y_hbm.at[pl.ds(base, _WPT)], yv)
    pltpu.sync_copy(z_hbm.at[pl.ds(base, _WPT)], zv)

    lane = lax.iota(jnp.int32, 16)
    is0 = lane == 0
    is1 = lane == 1

    def hw_body(c, carry):
        s = c * 16
        xx = xv[pl.ds(s, 16)]
        yy = yv[pl.ds(s, 16)]
        zz = zv[pl.ds(s, 16)]
        nz = -zz
        hh = 250.0 * (-yy) / nz + 112.0
        ww = 250.0 * xx / nz + 112.0
        hv[pl.ds(s, 16)] = jnp.minimum(jnp.maximum(hh, 0.0), 223.0)
        wv[pl.ds(s, 16)] = jnp.minimum(jnp.maximum(ww, 0.0), 223.0)
        return carry

    lax.fori_loop(0, _WPT // 16, hw_body, 0)

    tabs = (t0, t1, t2, t3)
    idxs = (i11, i21, i12, i22)

    def blk_body(b, carry):
        s = b * _BLK

        xx = xv[pl.ds(s, 16)]
        yy = yv[pl.ds(s, 16)]
        zz = zv[pl.ds(s, 16)]
        for j in range(16):
            cvec = jnp.where(is0, xx[j], jnp.where(is1, yy[j], zz[j]))
            ob[pl.ds(j * _OUTD, 16)] = cvec

        for lvl, (H, dim, scale, col0) in enumerate(_LEVELS):
            tab = tabs[lvl]
            q11, q21, q12, q22 = qbufs[lvl]
            hx = hv[pl.ds(s, 16)] * scale
            wy = wv[pl.ds(s, 16)] * scale
            x1i = hx.astype(jnp.int32)
            x1f = x1i.astype(jnp.float32)
            x2f = jnp.where(x1f == hx, x1f, x1f + 1.0)
            x2i = jnp.minimum(x2f.astype(jnp.int32), H - 1)
            y1i = wy.astype(jnp.int32)
            y1f = y1i.astype(jnp.float32)
            y2f = jnp.where(y1f == wy, y1f, y1f + 1.0)
            y2i = jnp.minimum(y2f.astype(jnp.int32), H - 1)
            dx2 = x2f - hx
            dx1 = hx - x1f
            dy2 = y2f - wy
            dy1 = wy - y1f
            i11[...] = x1i * H + y1i
            i21[...] = x2i * H + y1i
            i12[...] = x1i * H + y2i
            i22[...] = x2i * H + y2i
            w11v = dx2 * dy2
            w21v = dx1 * dy2
            w12v = dx2 * dy1
            w22v = dx1 * dy1

            cps = [
                pltpu.async_copy(tab.at[idx], q, sem)
                for idx, q in zip(idxs, qbufs[lvl])
            ]
            for cp in cps:
                cp.wait()

            for j in range(16):
                a, bw, cw, dw = w11v[j], w21v[j], w12v[j], w22v[j]
                jb = j * _OUTD + col0

                def ch_body(kk, carry3, j=j, a=a, bw=bw, cw=cw, dw=dw,
                            jb=jb, q11=q11, q21=q21, q12=q12, q22=q22):
                    d = pl.ds(kk * 16, 16)
                    ob[pl.ds(jb + kk * 16, 16)] = (
                        a * q11[j, d] + bw * q21[j, d]
                        + cw * q12[j, d] + dw * q22[j, d])
                    return carry3

                lax.fori_loop(0, dim // 16, ch_body, 0)

        pltpu.sync_copy(ob, out.at[pl.ds((base + s) * _OUTD, _BLK * _OUTD)])
        return carry

    lax.fori_loop(0, _WPT // _BLK, blk_body, 0)


@functools.cache
def _build_sc_kernel():
    mesh = plsc.VectorSubcoreMesh(
        core_axis_name="c", subcore_axis_name="s", num_cores=2, num_subcores=16
    )
    return functools.partial(
        pl.kernel,
        out_type=jax.ShapeDtypeStruct((_N * _OUTD,), jnp.float32),
        mesh=mesh,
        scratch_types=_scratch_types(),
        compiler_params=pltpu.CompilerParams(use_tc_tiling_on_sc=False),
    )(_sc_body)


def kernel(inputs, img_feat0, img_feat1, img_feat2, img_feat3):
    x = inputs[:, 0]
    y = inputs[:, 1]
    z = inputs[:, 2]
    t0 = img_feat0.reshape(56 * 56, 64)
    t1 = img_feat1.reshape(28 * 28, 128)
    t2 = img_feat2.reshape(14 * 14, 256)
    t3 = img_feat3.reshape(7 * 7, 512)
    flat = _build_sc_kernel()(x, y, z, t0, t1, t2, t3)
    return flat.reshape(_N, _OUTD)


# speedup vs baseline: 6.6457x; 3.7354x over previous
import functools

import jax
import jax.numpy as jnp
from jax import lax
from jax.experimental import pallas as pl
from jax.experimental.pallas import tpu as pltpu
from jax.experimental.pallas import tpu_sc as plsc

_N = 50000
_NW = 32
_WPT = 1568
_LEVELS = (
    (56, 64, 0.25, 64),
    (28, 128, 0.125, 32),
    (14, 256, 0.0625, 16),
    (7, 512, 1.0 / 32.0, 16),
)


def _scratch_types():
    t = [
        pltpu.VMEM((_WPT,), jnp.float32),
        pltpu.VMEM((_WPT,), jnp.float32),
        pltpu.VMEM((_WPT,), jnp.float32),
        pltpu.VMEM((_WPT,), jnp.float32),
        pltpu.VMEM((_WPT,), jnp.float32),
    ]
    for (_, dim, _, blk) in _LEVELS:
        t.extend([pltpu.VMEM((blk, dim), jnp.float32)] * 4)
        t.append(pltpu.VMEM((blk * dim // 128, 128), jnp.float32))
        t.extend([pltpu.VMEM((blk,), jnp.int32)] * 4)
        t.extend([pltpu.VMEM((blk,), jnp.float32)] * 4)
    t.append(pltpu.SemaphoreType.DMA)
    return t


def _worker_id():
    return lax.axis_index("s") * 2 + lax.axis_index("c")


def _sc_body(x_hbm, y_hbm, z_hbm, t0, t1, t2, t3, o0, o1, o2, o3, *scr):
    xv, yv, zv, hv, wv = scr[:5]
    per_level = []
    k = 5
    for _ in _LEVELS:
        per_level.append(scr[k:k + 13])
        k += 13
    sem = scr[k]

    wid = _worker_id()
    base = jnp.minimum(wid * _WPT, _N - _WPT)

    pltpu.sync_copy(x_hbm.at[pl.ds(base, _WPT)], xv)
    pltpu.sync_copy(y_hbm.at[pl.ds(base, _WPT)], yv)
    pltpu.sync_copy(z_hbm.at[pl.ds(base, _WPT)], zv)

    def hw_body(c, carry):
        s = c * 16
        xx = xv[pl.ds(s, 16)]
        yy = yv[pl.ds(s, 16)]
        zz = zv[pl.ds(s, 16)]
        nz = -zz
        hh = 250.0 * (-yy) / nz + 112.0
        ww = 250.0 * xx / nz + 112.0
        hv[pl.ds(s, 16)] = jnp.minimum(jnp.maximum(hh, 0.0), 223.0)
        wv[pl.ds(s, 16)] = jnp.minimum(jnp.maximum(ww, 0.0), 223.0)
        return carry

    lax.fori_loop(0, _WPT // 16, hw_body, 0)

    tabs = (t0, t1, t2, t3)
    outs = (o0, o1, o2, o3)
    for lvl, (H, dim, scale, blk) in enumerate(_LEVELS):
        tab, out = tabs[lvl], outs[lvl]
        q11, q21, q12, q22, st, i11, i21, i12, i22, w11, w21, w12, w22 = \
            per_level[lvl]
        nb = -(-_WPT // blk)

        def blk_body(b, carry, tab=tab, out=out, H=H, dim=dim, scale=scale,
                     blk=blk, q11=q11, q21=q21, q12=q12, q22=q22, st=st,
                     i11=i11, i21=i21, i12=i12, i22=i22,
                     w11=w11, w21=w21, w12=w12, w22=w22):
            pb = jnp.minimum(b * blk, _WPT - blk)

            def iw_body(c, carry2):
                s = pb + c * 16
                t = c * 16
                hx = hv[pl.ds(s, 16)] * scale
                wy = wv[pl.ds(s, 16)] * scale
                x1i = hx.astype(jnp.int32)
                x1f = x1i.astype(jnp.float32)
                x2f = jnp.where(x1f == hx, x1f, x1f + 1.0)
                x2i = jnp.minimum(x2f.astype(jnp.int32), H - 1)
                y1i = wy.astype(jnp.int32)
                y1f = y1i.astype(jnp.float32)
                y2f = jnp.where(y1f == wy, y1f, y1f + 1.0)
                y2i = jnp.minimum(y2f.astype(jnp.int32), H - 1)
                dx2 = x2f - hx
                dx1 = hx - x1f
                dy2 = y2f - wy
                dy1 = wy - y1f
                i11[pl.ds(t, 16)] = x1i * H + y1i
                i21[pl.ds(t, 16)] = x2i * H + y1i
                i12[pl.ds(t, 16)] = x1i * H + y2i
                i22[pl.ds(t, 16)] = x2i * H + y2i
                w11[pl.ds(t, 16)] = dx2 * dy2
                w21[pl.ds(t, 16)] = dx1 * dy2
                w12[pl.ds(t, 16)] = dx2 * dy1
                w22[pl.ds(t, 16)] = dx1 * dy1
                return carry2

            lax.fori_loop(0, blk // 16, iw_body, 0)

            c1 = pltpu.async_copy(tab.at[i11], q11, sem)
            c2 = pltpu.async_copy(tab.at[i21], q21, sem)
            c3 = pltpu.async_copy(tab.at[i12], q12, sem)
            c4 = pltpu.async_copy(tab.at[i22], q22, sem)
            c1.wait()
            c2.wait()
            c3.wait()
            c4.wait()

            def fma_body(g, carry2):
                p0 = g * 16
                a16 = w11[pl.ds(p0, 16)]
                b16 = w21[pl.ds(p0, 16)]
                c16 = w12[pl.ds(p0, 16)]
                d16 = w22[pl.ds(p0, 16)]
                for j in range(16):
                    a, bw, cw, dw = a16[j], b16[j], c16[j], d16[j]

                    def ch_body(kk, carry3, j=j, a=a, bw=bw, cw=cw, dw=dw):
                        p = p0 + j
                        d = pl.ds(kk * 16, 16)
                        v = (a * q11[p, d] + bw * q21[p, d]
                             + cw * q12[p, d] + dw * q22[p, d])
                        f = p * dim + kk * 16
                        st[lax.shift_right_logical(f, 7),
                           pl.ds(lax.bitwise_and(f, 127), 16)] = v
                        return carry3

                    lax.fori_loop(0, dim // 16, ch_body, 0)
                return carry2

            lax.fori_loop(0, blk // 16, fma_body, 0)

            rows = blk * dim // 128
            pltpu.sync_copy(
                st, out.at[pl.ds((base + pb) * dim // 128, rows)])
            return carry

        lax.fori_loop(0, nb, blk_body, 0)


@functools.cache
def _build_sc_kernel():
    mesh = plsc.VectorSubcoreMesh(
        core_axis_name="c", subcore_axis_name="s", num_cores=2, num_subcores=16
    )
    return functools.partial(
        pl.kernel,
        out_type=tuple(
            jax.ShapeDtypeStruct((_N * dim // 128, 128), jnp.float32)
            for (_, dim, _, _) in _LEVELS
        ),
        mesh=mesh,
        scratch_types=_scratch_types(),
        compiler_params=pltpu.CompilerParams(use_tc_tiling_on_sc=False),
    )(_sc_body)


def kernel(inputs, img_feat0, img_feat1, img_feat2, img_feat3):
    x = inputs[:, 0]
    y = inputs[:, 1]
    z = inputs[:, 2]
    t0 = img_feat0.reshape(56 * 56, 64)
    t1 = img_feat1.reshape(28 * 28, 128)
    t2 = img_feat2.reshape(14 * 14, 256)
    t3 = img_feat3.reshape(7 * 7, 512)
    o0, o1, o2, o3 = _build_sc_kernel()(x, y, z, t0, t1, t2, t3)
    return jnp.concatenate([
        inputs,
        o0.reshape(_N, 64),
        o1.reshape(_N, 128),
        o2.reshape(_N, 256),
        o3.reshape(_N, 512),
    ], axis=1)
